# baseline (speedup 1.0000x reference)
import jax
import jax.numpy as jnp
from jax.experimental import pallas as pl

_CONF = 0.25
_IOU = 0.45
_MAX_DET = 300
_NC = 80
_IMGSZ = 640.0
_K = 1500
_KPAD = 1536
_ROWT = 128
_TN = 2000


def _score_cls_kernel(scores_ref, smax_ref, cls_ref):
    s = scores_ref[0]
    pad = jnp.full((s.shape[0], 128 - s.shape[1]), -1.0, s.dtype)
    st = jnp.concatenate([s, pad], axis=1).T
    m = jnp.max(st, axis=0)
    a = jnp.argmax(st, axis=0)
    smax_ref[0, 0, 0] = jnp.where(m > _CONF, m, 0.0)
    cls_ref[0, 0, 0] = a.astype(jnp.float32)


def _suppress_kernel(ts_ref, x1c_ref, y1c_ref, x2c_ref, y2c_ref,
                     x1r_ref, y1r_ref, x2r_ref, y2r_ref, out_ref):
    cx1 = x1c_ref[0]
    cy1 = y1c_ref[0]
    cx2 = x2c_ref[0]
    cy2 = y2c_ref[0]
    carea = (cx2 - cx1) * (cy2 - cy1)
    col_id = jax.lax.broadcasted_iota(jnp.int32, (1, _KPAD), 1)

    def body(t, colmax):
        base = t * _ROWT
        rx1 = x1r_ref[0, pl.ds(base, _ROWT), :]
        ry1 = y1r_ref[0, pl.ds(base, _ROWT), :]
        rx2 = x2r_ref[0, pl.ds(base, _ROWT), :]
        ry2 = y2r_ref[0, pl.ds(base, _ROWT), :]
        rarea = (rx2 - rx1) * (ry2 - ry1)
        ix1 = jnp.maximum(rx1, cx1)
        iy1 = jnp.maximum(ry1, cy1)
        ix2 = jnp.minimum(rx2, cx2)
        iy2 = jnp.minimum(ry2, cy2)
        w = jnp.maximum(ix2 - ix1, 0.0)
        h = jnp.maximum(iy2 - iy1, 0.0)
        inter = w * h
        union = rarea + carea - inter
        iou = inter / (union + 1e-7)
        row_id = base + jax.lax.broadcasted_iota(jnp.int32, (_ROWT, 1), 0)
        masked = jnp.where(row_id < col_id, iou, 0.0)
        return jnp.maximum(colmax, jnp.max(masked, axis=0, keepdims=True))

    colmax = jax.lax.fori_loop(
        0, _KPAD // _ROWT, body, jnp.zeros((1, _KPAD), jnp.float32))
    ts = ts_ref[0]
    keep = (colmax < _IOU) & (ts > _CONF)
    out_ref[0] = jnp.where(keep, ts, -1.0)


def kernel(boxes, scores):
    B, N, C = scores.shape

    nb = N // _TN
    ospec = pl.BlockSpec((1, 1, 1, _TN), lambda b, n: (b, n, 0, 0))
    smax, clsf = pl.pallas_call(
        _score_cls_kernel,
        grid=(B, nb),
        in_specs=[pl.BlockSpec((1, _TN, C), lambda b, n: (b, n, 0))],
        out_specs=[ospec, ospec],
        out_shape=[jax.ShapeDtypeStruct((B, nb, 1, _TN), jnp.float32),
                   jax.ShapeDtypeStruct((B, nb, 1, _TN), jnp.float32)],
    )(scores)
    smax = smax.reshape(B, N)
    clsf = clsf.reshape(B, N)

    top_scores, top_idx = jax.lax.top_k(smax, _K)
    box = jnp.take_along_axis(boxes, top_idx[..., None], axis=1)
    c = jnp.take_along_axis(clsf, top_idx, axis=1)

    mult = 1.0 / _NC
    nms = mult * (box / _IMGSZ) + c[..., None] * mult

    pad = _KPAD - _K
    ts_p = jnp.pad(top_scores, ((0, 0), (0, pad)), constant_values=-1.0)
    nms_p = jnp.pad(nms, ((0, 0), (0, pad), (0, 0)))
    cols = [nms_p[:, None, :, i] for i in range(4)]
    rows = [nms_p[:, :, i][..., None] for i in range(4)]

    spec2 = pl.BlockSpec((1, 1, _KPAD), lambda b: (b, 0, 0))
    spec3 = pl.BlockSpec((1, _KPAD, 1), lambda b: (b, 0, 0))
    kept = pl.pallas_call(
        _suppress_kernel,
        grid=(B,),
        in_specs=[spec2, spec2, spec2, spec2, spec2,
                  spec3, spec3, spec3, spec3],
        out_specs=spec2,
        out_shape=jax.ShapeDtypeStruct((B, 1, _KPAD), jnp.float32),
    )(ts_p[:, None, :], *cols, *rows)

    kept = kept[:, 0, :_K]
    final_scores, final_idx = jax.lax.top_k(kept, _MAX_DET)
    valid = (final_scores > 0.0).astype(boxes.dtype)[..., None]
    fb = jnp.take_along_axis(box, final_idx[..., None], axis=1)
    fs = jnp.take_along_axis(top_scores, final_idx, axis=1)[..., None]
    fc = jnp.take_along_axis(c, final_idx, axis=1)[..., None]
    return jnp.concatenate([fb, fs, fc], axis=-1) * valid


# speedup vs baseline: 1.2337x; 1.2337x over previous
import jax
import jax.numpy as jnp
from jax.experimental import pallas as pl

_CONF = 0.25
_IOU = 0.45
_MAX_DET = 300
_NC = 80
_IMGSZ = 640.0
_K = 1500
_KPAD = 1536
_ROWT = 128
_TN = 2048


def _score_cls_kernel(scores_ref, smax_ref, cls_ref):
    s = scores_ref[0]
    m = jnp.max(s, axis=0)
    a = jnp.argmax(s, axis=0)
    smax_ref[0, 0, 0] = jnp.where(m > _CONF, m, 0.0)
    cls_ref[0, 0, 0] = a.astype(jnp.float32)


def _suppress_kernel(ts_ref, x1c_ref, y1c_ref, x2c_ref, y2c_ref,
                     x1r_ref, y1r_ref, x2r_ref, y2r_ref, out_ref):
    ax1 = x1r_ref[0]
    ay1 = y1r_ref[0]
    ax2 = x2r_ref[0]
    ay2 = y2r_ref[0]
    aarea = (ax2 - ax1) * (ay2 - ay1)
    row_id = jax.lax.broadcasted_iota(jnp.int32, (_KPAD, 1), 0)
    ts = ts_ref[0]
    nct = _KPAD // _ROWT
    for ct in range(nct):
        c0, c1 = ct * _ROWT, (ct + 1) * _ROWT
        cx1 = x1c_ref[0][:, c0:c1]
        cy1 = y1c_ref[0][:, c0:c1]
        cx2 = x2c_ref[0][:, c0:c1]
        cy2 = y2c_ref[0][:, c0:c1]
        carea = (cx2 - cx1) * (cy2 - cy1)
        rx1, ry1 = ax1[:c1], ay1[:c1]
        rx2, ry2 = ax2[:c1], ay2[:c1]
        ix1 = jnp.maximum(rx1, cx1)
        iy1 = jnp.maximum(ry1, cy1)
        ix2 = jnp.minimum(rx2, cx2)
        iy2 = jnp.minimum(ry2, cy2)
        w = jnp.maximum(ix2 - ix1, 0.0)
        h = jnp.maximum(iy2 - iy1, 0.0)
        inter = w * h
        union = aarea[:c1] + carea - inter
        iou = inter / (union + 1e-7)
        col_id = c0 + jax.lax.broadcasted_iota(jnp.int32, (1, _ROWT), 1)
        masked = jnp.where(row_id[:c1] < col_id, iou, 0.0)
        colmax = jnp.max(masked, axis=0, keepdims=True)
        tsc = ts[:, c0:c1]
        keep = (colmax < _IOU) & (tsc > _CONF)
        out_ref[0, :, c0:c1] = jnp.where(keep, tsc, -1.0)


def kernel(boxes, scores):
    B, N, C = scores.shape

    npad = -N % _TN
    st = jnp.transpose(scores, (0, 2, 1))
    st = jnp.pad(st, ((0, 0), (0, 0), (0, npad)), constant_values=-1.0)
    npb = st.shape[2]
    nb = npb // _TN
    ospec = pl.BlockSpec((1, 1, 1, _TN), lambda b, n: (b, n, 0, 0))
    smax, clsf = pl.pallas_call(
        _score_cls_kernel,
        grid=(B, nb),
        in_specs=[pl.BlockSpec((1, C, _TN), lambda b, n: (b, 0, n))],
        out_specs=[ospec, ospec],
        out_shape=[jax.ShapeDtypeStruct((B, nb, 1, _TN), jnp.float32),
                   jax.ShapeDtypeStruct((B, nb, 1, _TN), jnp.float32)],
    )(st)
    smax = smax.reshape(B, npb)
    clsf = clsf.reshape(B, npb)

    top_scores, top_idx = jax.lax.top_k(smax, _K)
    box = jnp.take_along_axis(boxes, top_idx[..., None], axis=1)
    c = jnp.take_along_axis(clsf, top_idx, axis=1)

    mult = 1.0 / _NC
    nms = mult * (box / _IMGSZ) + c[..., None] * mult

    pad = _KPAD - _K
    ts_p = jnp.pad(top_scores, ((0, 0), (0, pad)), constant_values=-1.0)
    nms_p = jnp.pad(nms, ((0, 0), (0, pad), (0, 0)))
    cols = [nms_p[:, None, :, i] for i in range(4)]
    rows = [nms_p[:, :, i][..., None] for i in range(4)]

    spec2 = pl.BlockSpec((1, 1, _KPAD), lambda b: (b, 0, 0))
    spec3 = pl.BlockSpec((1, _KPAD, 1), lambda b: (b, 0, 0))
    kept = pl.pallas_call(
        _suppress_kernel,
        grid=(B,),
        in_specs=[spec2, spec2, spec2, spec2, spec2,
                  spec3, spec3, spec3, spec3],
        out_specs=spec2,
        out_shape=jax.ShapeDtypeStruct((B, 1, _KPAD), jnp.float32),
    )(ts_p[:, None, :], *cols, *rows)

    kept = kept[:, 0, :_K]
    final_scores, final_idx = jax.lax.top_k(kept, _MAX_DET)
    valid = (final_scores > 0.0).astype(boxes.dtype)[..., None]
    fb = jnp.take_along_axis(box, final_idx[..., None], axis=1)
    fs = jnp.take_along_axis(top_scores, final_idx, axis=1)[..., None]
    fc = jnp.take_along_axis(c, final_idx, axis=1)[..., None]
    return jnp.concatenate([fb, fs, fc], axis=-1) * valid
